# baseline (speedup 1.0000x reference)
import functools

import jax
import jax.numpy as jnp
from jax import lax
from jax.experimental import pallas as pl
from jax.experimental.pallas import tpu as pltpu
from jax.experimental.pallas import tpu_sc as plsc

H = 768
I = 1024
E = 64
N = 2048
M = 128
TOT = 96
ROWS = TOT * M
NW = 32
CHUNK = N // NW
LANES = 16
WL = 128


def _router_body(x_ref, g_ref, pos0_ref, pos1_ref, w0_ref, w1_ref, meta_ref):
    x = x_ref[...]
    g = g_ref[...]
    logits = lax.dot_general(x, g, (((1,), (1,)), ((), ())),
                             preferred_element_type=jnp.float32)
    col = lax.broadcasted_iota(jnp.int32, (N, E), 1)
    m0 = jnp.max(logits, axis=1, keepdims=True)
    am0 = jnp.min(jnp.where(logits == m0, col, E), axis=1, keepdims=True)
    neg = jnp.float32(-3.4e38)
    logits1 = jnp.where(col == am0, neg, logits)
    m1 = jnp.max(logits1, axis=1, keepdims=True)
    am1 = jnp.min(jnp.where(logits1 == m1, col, E), axis=1, keepdims=True)
    wt0 = jax.nn.sigmoid(m0 - m1)
    wt1 = 1.0 - wt0

    oh0 = (col == am0).astype(jnp.float32)
    oh1 = (col == am1).astype(jnp.float32)
    oh = oh0 + oh1

    tri = (lax.broadcasted_iota(jnp.int32, (M, M), 1)
           < lax.broadcasted_iota(jnp.int32, (M, M), 0)).astype(jnp.float32)
    blocks = []
    bsums = []
    for b in range(N // M):
        ob = oh[b * M:(b + 1) * M, :]
        blocks.append(lax.dot_general(tri, ob, (((1,), (0,)), ((), ())),
                                      preferred_element_type=jnp.float32))
        bsums.append(jnp.sum(ob, axis=0, keepdims=True))
    bsum = jnp.concatenate(bsums, axis=0)
    nb = N // M
    trib = (lax.broadcasted_iota(jnp.int32, (nb, nb), 1)
            < lax.broadcasted_iota(jnp.int32, (nb, nb), 0)).astype(jnp.float32)
    boff = lax.dot_general(trib, bsum, (((1,), (0,)), ((), ())),
                           preferred_element_type=jnp.float32)
    excl = jnp.concatenate(
        [blocks[b] + boff[b:b + 1, :] for b in range(nb)], axis=0)

    counts = jnp.sum(bsum, axis=0, keepdims=True)
    ntiles = jnp.floor((counts + (M - 1)) / M)
    trie = (lax.broadcasted_iota(jnp.int32, (E, E), 0)
            <= lax.broadcasted_iota(jnp.int32, (E, E), 1)).astype(jnp.float32)
    tilecum = lax.dot_general(ntiles, trie, (((1,), (0,)), ((), ())),
                              preferred_element_type=jnp.float32)
    offsets = (tilecum - ntiles) * M
    dest = offsets + excl
    pos0 = jnp.sum(oh0 * dest, axis=1).astype(jnp.int32)
    pos1 = jnp.sum(oh1 * dest, axis=1).astype(jnp.int32)
    pos0_ref[...] = pos0.reshape(N // M, M)
    pos1_ref[...] = pos1.reshape(N // M, M)
    w0_ref[...] = jnp.broadcast_to(wt0, (N, WL))
    w1_ref[...] = jnp.broadcast_to(wt1, (N, WL))

    total = jnp.sum(ntiles)
    ii = lax.broadcasted_iota(jnp.int32, (M, E), 0)
    tcb = jnp.broadcast_to(tilecum.astype(jnp.int32), (M, E))
    te = jnp.sum((ii >= tcb).astype(jnp.int32), axis=1)
    meta_ref[0:1, :] = te.reshape(1, M)
    meta_ref[1:2, :] = jnp.broadcast_to(
        total.astype(jnp.int32).reshape(1, 1), (1, M))


def _router(x, gate_w):
    f32 = jnp.float32
    return pl.pallas_call(
        _router_body,
        out_shape=(
            jax.ShapeDtypeStruct((N // M, M), jnp.int32),
            jax.ShapeDtypeStruct((N // M, M), jnp.int32),
            jax.ShapeDtypeStruct((N, WL), f32),
            jax.ShapeDtypeStruct((N, WL), f32),
            jax.ShapeDtypeStruct((8, M), jnp.int32),
        ),
    )(x, gate_w)


def _scatter_body(x_hbm, pos0_hbm, pos1_hbm, w0_hbm, w1_hbm, out_hbm,
                  wout_hbm, idx0_v, idx1_v, rows_v, wr0_v, wr1_v,
                  sem0, sem1, sem2, sem3, sem4):
    info = plsc.get_sparse_core_info()
    wid = lax.axis_index("s") * info.num_cores + lax.axis_index("c")
    base = wid * CHUNK
    cx = pltpu.async_copy(x_hbm.at[pl.ds(base, CHUNK)], rows_v, sem4)
    c0 = pltpu.async_copy(pos0_hbm.at[pl.ds(base, CHUNK)], idx0_v, sem0)
    c1 = pltpu.async_copy(pos1_hbm.at[pl.ds(base, CHUNK)], idx1_v, sem1)
    c2 = pltpu.async_copy(w0_hbm.at[pl.ds(base, CHUNK)], wr0_v, sem2)
    c3 = pltpu.async_copy(w1_hbm.at[pl.ds(base, CHUNK)], wr1_v, sem3)
    c0.wait()
    c1.wait()
    c2.wait()
    c3.wait()
    s2 = pltpu.async_copy(wr0_v, wout_hbm.at[idx0_v], sem2)
    s3 = pltpu.async_copy(wr1_v, wout_hbm.at[idx1_v], sem3)
    cx.wait()
    s0 = pltpu.async_copy(rows_v, out_hbm.at[idx0_v], sem0)
    s1 = pltpu.async_copy(rows_v, out_hbm.at[idx1_v], sem1)
    s0.wait()
    s1.wait()
    s2.wait()
    s3.wait()


def _scatter(x, pos0, pos1, w0b, w1b):
    mesh = plsc.VectorSubcoreMesh(core_axis_name="c", subcore_axis_name="s")
    return pl.kernel(
        _scatter_body,
        out_type=(
            jax.ShapeDtypeStruct((ROWS, H), jnp.float32),
            jax.ShapeDtypeStruct((ROWS, WL), jnp.float32),
        ),
        mesh=mesh,
        scratch_types=[
            pltpu.VMEM((CHUNK,), jnp.int32),
            pltpu.VMEM((CHUNK,), jnp.int32),
            pltpu.VMEM((CHUNK, H), jnp.float32),
            pltpu.VMEM((CHUNK, WL), jnp.float32),
            pltpu.VMEM((CHUNK, WL), jnp.float32),
            pltpu.SemaphoreType.DMA,
            pltpu.SemaphoreType.DMA,
            pltpu.SemaphoreType.DMA,
            pltpu.SemaphoreType.DMA,
            pltpu.SemaphoreType.DMA,
        ],
    )(x, pos0, pos1, w0b, w1b)


def _mlp_body(te_ref, tot_ref, x_ref, w1_ref, w3_ref, w2_ref, wt_ref, y_ref):
    i = pl.program_id(0)

    @pl.when(i < tot_ref[0])
    def _():
        x = x_ref[...]
        a = lax.dot_general(x, w1_ref[0], (((1,), (1,)), ((), ())),
                            preferred_element_type=jnp.float32)
        b = lax.dot_general(x, w3_ref[0], (((1,), (1,)), ((), ())),
                            preferred_element_type=jnp.float32)
        h = a * jax.nn.sigmoid(a) * b
        y = lax.dot_general(h, w2_ref[0], (((1,), (1,)), ((), ())),
                            preferred_element_type=jnp.float32)
        y_ref[...] = y * wt_ref[:, 0:1]


def _grouped_mlp(x_sorted, w1, w3, w2, w_sorted, te, tot):
    def clamp(i, te_ref, tot_ref):
        return jnp.minimum(i, tot_ref[0] - 1)

    grid_spec = pltpu.PrefetchScalarGridSpec(
        num_scalar_prefetch=2,
        grid=(TOT,),
        in_specs=[
            pl.BlockSpec((M, H), lambda i, te_ref, tot_ref:
                         (jnp.minimum(i, tot_ref[0] - 1), 0)),
            pl.BlockSpec((1, I, H), lambda i, te_ref, tot_ref:
                         (te_ref[jnp.minimum(i, tot_ref[0] - 1)], 0, 0)),
            pl.BlockSpec((1, I, H), lambda i, te_ref, tot_ref:
                         (te_ref[jnp.minimum(i, tot_ref[0] - 1)], 0, 0)),
            pl.BlockSpec((1, H, I), lambda i, te_ref, tot_ref:
                         (te_ref[jnp.minimum(i, tot_ref[0] - 1)], 0, 0)),
            pl.BlockSpec((M, WL), lambda i, te_ref, tot_ref:
                         (jnp.minimum(i, tot_ref[0] - 1), 0)),
        ],
        out_specs=pl.BlockSpec((M, H), lambda i, te_ref, tot_ref:
                               (jnp.minimum(i, tot_ref[0] - 1), 0)),
    )
    return pl.pallas_call(
        _mlp_body,
        grid_spec=grid_spec,
        out_shape=jax.ShapeDtypeStruct((ROWS, H), jnp.float32),
    )(te, tot, x_sorted, w1, w3, w2, w_sorted)


HALF = CHUNK // 2


QUAR = CHUNK // 4


def _combine_body(y_hbm, pos0_hbm, pos1_hbm, out_hbm, idx0_v, idx1_v,
                  buf, buf2, semi, semq0, semq1, semq2, semq3):
    info = plsc.get_sparse_core_info()
    wid = lax.axis_index("s") * info.num_cores + lax.axis_index("c")
    base = wid * CHUNK
    semq = [semq0, semq1, semq2, semq3]
    ci0 = pltpu.async_copy(pos0_hbm.at[pl.ds(base, CHUNK)], idx0_v, semi)
    ci1 = pltpu.async_copy(pos1_hbm.at[pl.ds(base, CHUNK)], idx1_v, semi)
    ci0.wait()
    ci1.wait()
    gathers = []
    for q in range(4):
        sl = pl.ds(q * QUAR, QUAR)
        g0 = pltpu.async_copy(y_hbm.at[idx0_v.at[sl]], buf.at[sl], semq[q])
        g1 = pltpu.async_copy(y_hbm.at[idx1_v.at[sl]], buf2.at[sl], semq[q])
        gathers.append((g0, g1))

    def token_body(t, carry):
        for gidx in range(H // LANES):
            sl = pl.ds(gidx * LANES, LANES)
            buf[t, sl] = buf[t, sl] + buf2[t, sl]
        return carry

    stores = []
    for q in range(4):
        gathers[q][0].wait()
        gathers[q][1].wait()
        lax.fori_loop(q * QUAR, (q + 1) * QUAR, token_body, 0)
        stores.append(pltpu.async_copy(
            buf.at[pl.ds(q * QUAR, QUAR)],
            out_hbm.at[pl.ds(base + q * QUAR, QUAR)], semi))
    for st in stores:
        st.wait()


def _combine(y, pos0, pos1):
    mesh = plsc.VectorSubcoreMesh(core_axis_name="c", subcore_axis_name="s")
    return pl.kernel(
        _combine_body,
        out_type=jax.ShapeDtypeStruct((N, H), jnp.float32),
        mesh=mesh,
        scratch_types=[
            pltpu.VMEM((CHUNK,), jnp.int32),
            pltpu.VMEM((CHUNK,), jnp.int32),
            pltpu.VMEM((CHUNK, H), jnp.float32),
            pltpu.VMEM((CHUNK, H), jnp.float32),
            pltpu.SemaphoreType.DMA,
            pltpu.SemaphoreType.DMA,
            pltpu.SemaphoreType.DMA,
            pltpu.SemaphoreType.DMA,
            pltpu.SemaphoreType.DMA,
        ],
    )(y, pos0, pos1)


@jax.jit
def kernel(hidden_states, gate_w, w1, w2, w3):
    pos0_2d, pos1_2d, w0_2d, w1_2d, meta = _router(hidden_states, gate_w)
    pos0 = pos0_2d.reshape(N)
    pos1 = pos1_2d.reshape(N)
    te = meta[0]
    tot = meta[1, 0:1]
    x_sorted, w_sorted = _scatter(hidden_states, pos0, pos1, w0_2d, w1_2d)
    y = _grouped_mlp(x_sorted, w1, w3, w2, w_sorted, te, tot)
    return _combine(y, pos0, pos1)


# speedup vs baseline: 1.0166x; 1.0166x over previous
import functools

import jax
import jax.numpy as jnp
from jax import lax
from jax.experimental import pallas as pl
from jax.experimental.pallas import tpu as pltpu
from jax.experimental.pallas import tpu_sc as plsc

H = 768
I = 1024
E = 64
N = 2048
M = 128
TOT = 96
ROWS = TOT * M
NW = 32
CHUNK = N // NW
LANES = 16
WL = 128


def _router_body(x_ref, g_ref, pos0_ref, pos1_ref, w0_ref, w1_ref, meta_ref):
    x = x_ref[...]
    g = g_ref[...]
    logits = lax.dot_general(x, g, (((1,), (1,)), ((), ())),
                             preferred_element_type=jnp.float32)
    col = lax.broadcasted_iota(jnp.int32, (N, E), 1)
    m0 = jnp.max(logits, axis=1, keepdims=True)
    am0 = jnp.min(jnp.where(logits == m0, col, E), axis=1, keepdims=True)
    neg = jnp.float32(-3.4e38)
    logits1 = jnp.where(col == am0, neg, logits)
    m1 = jnp.max(logits1, axis=1, keepdims=True)
    am1 = jnp.min(jnp.where(logits1 == m1, col, E), axis=1, keepdims=True)
    wt0 = jax.nn.sigmoid(m0 - m1)
    wt1 = 1.0 - wt0

    oh0 = (col == am0).astype(jnp.float32)
    oh1 = (col == am1).astype(jnp.float32)
    oh = oh0 + oh1

    tri = (lax.broadcasted_iota(jnp.int32, (M, M), 1)
           < lax.broadcasted_iota(jnp.int32, (M, M), 0)).astype(jnp.float32)
    blocks = []
    bsums = []
    for b in range(N // M):
        ob = oh[b * M:(b + 1) * M, :]
        blocks.append(lax.dot_general(tri, ob, (((1,), (0,)), ((), ())),
                                      preferred_element_type=jnp.float32))
        bsums.append(jnp.sum(ob, axis=0, keepdims=True))
    bsum = jnp.concatenate(bsums, axis=0)
    nb = N // M
    trib = (lax.broadcasted_iota(jnp.int32, (nb, nb), 1)
            < lax.broadcasted_iota(jnp.int32, (nb, nb), 0)).astype(jnp.float32)
    boff = lax.dot_general(trib, bsum, (((1,), (0,)), ((), ())),
                           preferred_element_type=jnp.float32)
    excl = jnp.concatenate(
        [blocks[b] + boff[b:b + 1, :] for b in range(nb)], axis=0)

    counts = jnp.sum(bsum, axis=0, keepdims=True)
    ntiles = jnp.floor((counts + (M - 1)) / M)
    trie = (lax.broadcasted_iota(jnp.int32, (E, E), 0)
            <= lax.broadcasted_iota(jnp.int32, (E, E), 1)).astype(jnp.float32)
    tilecum = lax.dot_general(ntiles, trie, (((1,), (0,)), ((), ())),
                              preferred_element_type=jnp.float32)
    offsets = (tilecum - ntiles) * M
    dest = offsets + excl
    pos0 = jnp.sum(oh0 * dest, axis=1).astype(jnp.int32)
    pos1 = jnp.sum(oh1 * dest, axis=1).astype(jnp.int32)
    pos0_ref[...] = pos0.reshape(N // M, M)
    pos1_ref[...] = pos1.reshape(N // M, M)
    w0_ref[...] = jnp.broadcast_to(wt0, (N, LANES))
    w1_ref[...] = jnp.broadcast_to(wt1, (N, LANES))

    total = jnp.sum(ntiles)
    ii = lax.broadcasted_iota(jnp.int32, (M, E), 0)
    tcb = jnp.broadcast_to(tilecum.astype(jnp.int32), (M, E))
    te = jnp.sum((ii >= tcb).astype(jnp.int32), axis=1)
    meta_ref[0:1, :] = te.reshape(1, M)
    meta_ref[1:2, :] = jnp.broadcast_to(
        total.astype(jnp.int32).reshape(1, 1), (1, M))


def _router(x, gate_w):
    f32 = jnp.float32
    return pl.pallas_call(
        _router_body,
        out_shape=(
            jax.ShapeDtypeStruct((N // M, M), jnp.int32),
            jax.ShapeDtypeStruct((N // M, M), jnp.int32),
            jax.ShapeDtypeStruct((N, LANES), f32),
            jax.ShapeDtypeStruct((N, LANES), f32),
            jax.ShapeDtypeStruct((8, M), jnp.int32),
        ),
    )(x, gate_w)


def _scatter_body(x_hbm, pos0_hbm, pos1_hbm, out_hbm, idx0_v, idx1_v, rows_v,
                  sem0, sem1, sem2):
    info = plsc.get_sparse_core_info()
    wid = lax.axis_index("s") * info.num_cores + lax.axis_index("c")
    base = wid * CHUNK
    cx = pltpu.async_copy(x_hbm.at[pl.ds(base, CHUNK)], rows_v, sem2)
    c0 = pltpu.async_copy(pos0_hbm.at[pl.ds(base, CHUNK)], idx0_v, sem0)
    c1 = pltpu.async_copy(pos1_hbm.at[pl.ds(base, CHUNK)], idx1_v, sem1)
    c0.wait()
    c1.wait()
    cx.wait()
    s0 = pltpu.async_copy(rows_v, out_hbm.at[idx0_v], sem0)
    s1 = pltpu.async_copy(rows_v, out_hbm.at[idx1_v], sem1)
    s0.wait()
    s1.wait()


def _scatter(x, pos0, pos1):
    mesh = plsc.VectorSubcoreMesh(core_axis_name="c", subcore_axis_name="s")
    return pl.kernel(
        _scatter_body,
        out_type=jax.ShapeDtypeStruct((ROWS, H), jnp.float32),
        mesh=mesh,
        scratch_types=[
            pltpu.VMEM((CHUNK,), jnp.int32),
            pltpu.VMEM((CHUNK,), jnp.int32),
            pltpu.VMEM((CHUNK, H), jnp.float32),
            pltpu.SemaphoreType.DMA,
            pltpu.SemaphoreType.DMA,
            pltpu.SemaphoreType.DMA,
        ],
    )(x, pos0, pos1)


def _mlp_body(te_ref, tot_ref, x_ref, w1_ref, w3_ref, w2_ref, y_ref):
    i = pl.program_id(0)

    @pl.when(i < tot_ref[0])
    def _():
        x = x_ref[...]
        a = lax.dot_general(x, w1_ref[0], (((1,), (1,)), ((), ())),
                            preferred_element_type=jnp.float32)
        b = lax.dot_general(x, w3_ref[0], (((1,), (1,)), ((), ())),
                            preferred_element_type=jnp.float32)
        h = a * jax.nn.sigmoid(a) * b
        y_ref[...] = lax.dot_general(h, w2_ref[0], (((1,), (1,)), ((), ())),
                                     preferred_element_type=jnp.float32)


def _grouped_mlp(x_sorted, w1, w3, w2, te, tot):
    def clamp(i, te_ref, tot_ref):
        return jnp.minimum(i, tot_ref[0] - 1)

    grid_spec = pltpu.PrefetchScalarGridSpec(
        num_scalar_prefetch=2,
        grid=(TOT,),
        in_specs=[
            pl.BlockSpec((M, H), lambda i, te_ref, tot_ref:
                         (jnp.minimum(i, tot_ref[0] - 1), 0)),
            pl.BlockSpec((1, I, H), lambda i, te_ref, tot_ref:
                         (te_ref[jnp.minimum(i, tot_ref[0] - 1)], 0, 0)),
            pl.BlockSpec((1, I, H), lambda i, te_ref, tot_ref:
                         (te_ref[jnp.minimum(i, tot_ref[0] - 1)], 0, 0)),
            pl.BlockSpec((1, H, I), lambda i, te_ref, tot_ref:
                         (te_ref[jnp.minimum(i, tot_ref[0] - 1)], 0, 0)),
        ],
        out_specs=pl.BlockSpec((M, H), lambda i, te_ref, tot_ref:
                               (jnp.minimum(i, tot_ref[0] - 1), 0)),
    )
    return pl.pallas_call(
        _mlp_body,
        grid_spec=grid_spec,
        out_shape=jax.ShapeDtypeStruct((ROWS, H), jnp.float32),
    )(te, tot, x_sorted, w1, w3, w2)


HALF = CHUNK // 2


QUAR = CHUNK // 4


def _combine_body(y_hbm, pos0_hbm, pos1_hbm, w0_hbm, w1_hbm, out_hbm,
                  idx0_v, idx1_v, w0_v, w1_v,
                  buf, buf2, semi, semq0, semq1, semq2, semq3):
    info = plsc.get_sparse_core_info()
    wid = lax.axis_index("s") * info.num_cores + lax.axis_index("c")
    base = wid * CHUNK
    semq = [semq0, semq1, semq2, semq3]
    ci0 = pltpu.async_copy(pos0_hbm.at[pl.ds(base, CHUNK)], idx0_v, semi)
    ci1 = pltpu.async_copy(pos1_hbm.at[pl.ds(base, CHUNK)], idx1_v, semi)
    cw0 = pltpu.async_copy(w0_hbm.at[pl.ds(base, CHUNK)], w0_v, semi)
    cw1 = pltpu.async_copy(w1_hbm.at[pl.ds(base, CHUNK)], w1_v, semi)
    ci0.wait()
    ci1.wait()
    gathers = []
    for q in range(4):
        sl = pl.ds(q * QUAR, QUAR)
        g0 = pltpu.async_copy(y_hbm.at[idx0_v.at[sl]], buf.at[sl], semq[q])
        g1 = pltpu.async_copy(y_hbm.at[idx1_v.at[sl]], buf2.at[sl], semq[q])
        gathers.append((g0, g1))

    def token_body(t, carry):
        w0s = w0_v[t, :]
        w1s = w1_v[t, :]
        for gidx in range(H // LANES):
            sl = pl.ds(gidx * LANES, LANES)
            buf[t, sl] = buf[t, sl] * w0s + buf2[t, sl] * w1s
        return carry

    cw0.wait()
    cw1.wait()
    stores = []
    for q in range(4):
        gathers[q][0].wait()
        gathers[q][1].wait()
        lax.fori_loop(q * QUAR, (q + 1) * QUAR, token_body, 0)
        stores.append(pltpu.async_copy(
            buf.at[pl.ds(q * QUAR, QUAR)],
            out_hbm.at[pl.ds(base + q * QUAR, QUAR)], semi))
    for st in stores:
        st.wait()


def _combine(y, pos0, pos1, w0b, w1b):
    mesh = plsc.VectorSubcoreMesh(core_axis_name="c", subcore_axis_name="s")
    return pl.kernel(
        _combine_body,
        out_type=jax.ShapeDtypeStruct((N, H), jnp.float32),
        mesh=mesh,
        scratch_types=[
            pltpu.VMEM((CHUNK,), jnp.int32),
            pltpu.VMEM((CHUNK,), jnp.int32),
            pltpu.VMEM((CHUNK, LANES), jnp.float32),
            pltpu.VMEM((CHUNK, LANES), jnp.float32),
            pltpu.VMEM((CHUNK, H), jnp.float32),
            pltpu.VMEM((CHUNK, H), jnp.float32),
            pltpu.SemaphoreType.DMA,
            pltpu.SemaphoreType.DMA,
            pltpu.SemaphoreType.DMA,
            pltpu.SemaphoreType.DMA,
            pltpu.SemaphoreType.DMA,
        ],
    )(y, pos0, pos1, w0b, w1b)


@jax.jit
def kernel(hidden_states, gate_w, w1, w2, w3):
    pos0_2d, pos1_2d, w0_2d, w1_2d, meta = _router(hidden_states, gate_w)
    pos0 = pos0_2d.reshape(N)
    pos1 = pos1_2d.reshape(N)
    te = meta[0]
    tot = meta[1, 0:1]
    x_sorted = _scatter(hidden_states, pos0, pos1)
    y = _grouped_mlp(x_sorted, w1, w3, w2, te, tot)
    return _combine(y, pos0, pos1, w0_2d, w1_2d)
